# baseline (speedup 1.0000x reference)
import functools

import jax
import jax.numpy as jnp
from jax import lax
from jax.experimental import pallas as pl
from jax.experimental.pallas import tpu as pltpu
from jax.experimental.pallas import tpu_sc as plsc

N_NODES = 10000
N_EDGES = 320000
D_IN = 128
D_HID = 128
N_CLASSES = 40
D2P = 48

NC = 2
NS = 16
NW = NC * NS
K = 128
RPW = 80
E_PAD = NW * RPW * K
NROWS = NW * RPW
N_ACC = 10240
CR0 = 160
CR1 = 160 - CR0

_MESH = plsc.VectorSubcoreMesh(core_axis_name="c", subcore_axis_name="s")


def _make_sc_degree():
    @functools.partial(
        pl.kernel,
        out_type=jax.ShapeDtypeStruct((NC * N_ACC,), jnp.float32),
        mesh=_MESH,
        scratch_types=[
            pltpu.VMEM((RPW, K), jnp.int32),
            pltpu.VMEM((RPW, K), jnp.float32),
            pltpu.VMEM((640,), jnp.float32),
            pltpu.VMEM_SHARED((N_ACC,), jnp.float32),
        ],
    )
    def deg_kernel(dst_hbm, ew_hbm, deg_hbm, dstv, ewv, zv, deg_sh):
        c = lax.axis_index("c")
        s = lax.axis_index("s")
        w = c * NS + s
        for i in range(640 // 16):
            zv[pl.ds(i * 16, 16)] = jnp.zeros((16,), jnp.float32)
        pltpu.sync_copy(zv, deg_sh.at[pl.ds(s * 640, 640)])
        plsc.subcore_barrier()
        pltpu.sync_copy(dst_hbm.at[pl.ds(w * RPW, RPW)], dstv)
        pltpu.sync_copy(ew_hbm.at[pl.ds(w * RPW, RPW)], ewv)

        def body(j, carry):
            pltpu.sync_copy(ewv.at[j], deg_sh.at[dstv.at[j]], add=True)
            return carry

        lax.fori_loop(0, RPW, body, 0)
        plsc.subcore_barrier()
        pltpu.sync_copy(deg_sh.at[pl.ds(s * 640, 640)],
                        deg_hbm.at[pl.ds(c * N_ACC + s * 640, 640)])

    return deg_kernel


def _make_sc_agg(d):
    @functools.partial(
        pl.kernel,
        out_type=jax.ShapeDtypeStruct((1, N_ACC, d), jnp.float32),
        mesh=_MESH,
        scratch_types=[
            pltpu.VMEM((8, 2, K), jnp.int32),
            pltpu.VMEM((8, K), jnp.float32),
            pltpu.VMEM((8, 2, K), jnp.int32),
            pltpu.VMEM((8, K), jnp.float32),
            pltpu.VMEM((K, d), jnp.float32),
            pltpu.VMEM((K, d), jnp.float32),
            pltpu.VMEM_SHARED((N_ACC, d), jnp.float32),
            pltpu.SemaphoreType.DMA,
            pltpu.SemaphoreType.DMA,
            pltpu.SemaphoreType.DMA,
            pltpu.SemaphoreType.DMA,
            pltpu.SemaphoreType.DMA,
            pltpu.SemaphoreType.DMA,
        ],
    )
    def agg_kernel(h_hbm, e3_hbm, ew_hbm, acc_hbm,
                   e3A, ewA, e3B, ewB, rows0, rows1, acc_sh,
                   g0, g1, s0, s1, iA, iB):
        c = lax.axis_index("c")
        s = lax.axis_index("s")
        base_w = s * CR0
        nsup = CR0 // 16

        def zbody(i, carry):
            for t in range(d // 16):
                rows0[i, pl.ds(t * 16, 16)] = jnp.zeros((16,), jnp.float32)
            return carry

        @pl.when(c == 0)
        def _():
            lax.fori_loop(0, K, zbody, 0)
            for m in range(640 // K):
                pltpu.sync_copy(rows0, acc_sh.at[pl.ds(s * 640 + m * K, K)])
            plsc.subcore_barrier()

        def scale(buf, ewb, j):
            def group(g, carry2):
                w16 = ewb[j, pl.ds(g * 16, 16)]
                for l in range(16):
                    sc = w16[l]
                    k = g * 16 + l
                    for t in range(d // 16):
                        buf[k, pl.ds(t * 16, 16)] = (
                            buf[k, pl.ds(t * 16, 16)] * sc)
                return carry2

            lax.fori_loop(0, K // 16, group, 0)

        def wait_gather(buf, sem):
            pltpu.make_async_copy(h_hbm.at[e3A.at[0, 0]], buf, sem).wait()

        def issue_idx(base, e3b, ewb, sem):
            pltpu.async_copy(e3_hbm.at[pl.ds(base, 8)], e3b, sem)
            pltpu.async_copy(ew_hbm.at[pl.ds(base, 8)], ewb, sem)

        def wait_idx(e3b, ewb, sem):
            pltpu.make_async_copy(e3_hbm.at[pl.ds(0, 8)], e3b, sem).wait()
            pltpu.make_async_copy(ew_hbm.at[pl.ds(0, 8)], ewb, sem).wait()

        def run_block(e3b, ewb, e3n, ewn, sem_n, base_next2, load_next2,
                      has_next):
            def pair(p, carry1):
                j0 = 2 * p
                j1 = 2 * p + 1
                wait_gather(rows0, g0)
                scale(rows0, ewb, j0)
                d0 = pltpu.async_copy(rows0, acc_sh.at[e3b.at[j0, 1]], s0,
                                      add=True)
                wait_gather(rows1, g1)
                scale(rows1, ewb, j1)
                d1 = pltpu.async_copy(rows1, acc_sh.at[e3b.at[j1, 1]], s1,
                                      add=True)
                d0.wait()

                @pl.when(p < 3)
                def _():
                    pltpu.async_copy(h_hbm.at[e3b.at[j0 + 2, 0]], rows0, g0)

                d1.wait()

                @pl.when(p < 3)
                def _():
                    pltpu.async_copy(h_hbm.at[e3b.at[j1 + 2, 0]], rows1, g1)

                @pl.when((p == 3) & has_next)
                def _():
                    wait_idx(e3n, ewn, sem_n)
                    pltpu.async_copy(h_hbm.at[e3n.at[0, 0]], rows0, g0)
                    pltpu.async_copy(h_hbm.at[e3n.at[1, 0]], rows1, g1)

                @pl.when((p == 3) & load_next2)
                def _():
                    issue_idx(base_next2, e3b, ewb,
                              iA if e3b is e3A else iB)

                return carry1

            lax.fori_loop(0, 4, pair, 0)

        def sup(t, carry):
            base = base_w + t * 16
            not_last = t < nsup - 1
            run_block(e3A, ewA, e3B, ewB, iB, base + 16, not_last,
                      jnp.bool_(True))
            run_block(e3B, ewB, e3A, ewA, iA, base + 24, not_last, not_last)
            return carry

        @pl.when(c == 0)
        def _():
            issue_idx(base_w, e3A, ewA, iA)
            wait_idx(e3A, ewA, iA)
            issue_idx(base_w + 8, e3B, ewB, iB)
            pltpu.async_copy(h_hbm.at[e3A.at[0, 0]], rows0, g0)
            pltpu.async_copy(h_hbm.at[e3A.at[1, 0]], rows1, g1)
            lax.fori_loop(0, nsup, sup, 0)
            plsc.subcore_barrier()
            pltpu.sync_copy(acc_sh.at[pl.ds(s * 640, 640)],
                            acc_hbm.at[0, pl.ds(s * 640, 640)])

    return agg_kernel


_sc_degree = _make_sc_degree()
_sc_agg128 = _make_sc_agg(D_HID)


def _dis_of(degp_blk):
    deg = degp_blk[0, :] + degp_blk[1, :] + 1.0
    return lax.rsqrt(deg)


def _tc_mm1_body(x_ref, w_ref, degp_ref, out_ref):
    dis = _dis_of(degp_ref[...])
    h = jnp.dot(x_ref[...], w_ref[...], preferred_element_type=jnp.float32)
    out_ref[...] = h * dis[:, None]


def _tc_mid_body(acc_ref, h1s_ref, degp_ref, b1_ref, out_ref):
    dis = _dis_of(degp_ref[...])
    a = acc_ref[0] + h1s_ref[...]
    z = jnp.maximum(a * dis[:, None] + b1_ref[...], 0.0)
    out_ref[...] = z * dis[:, None]


def _tc_final_body(acc_ref, zs_ref, degp_ref, w2_ref, b2_ref, out_ref):
    dis = _dis_of(degp_ref[...])
    a = (acc_ref[0] + zs_ref[...]) * dis[:, None]
    out_ref[...] = jnp.dot(a, w2_ref[...],
                           preferred_element_type=jnp.float32) + b2_ref[...]


_RB = 1024


def kernel(x, edge_index, edge_weight, W1, b1, W2, b2):
    src = edge_index[0].astype(jnp.int32)
    dst = edge_index[1].astype(jnp.int32)
    pad = E_PAD - N_EDGES
    src2d = jnp.pad(src, (0, pad)).reshape(NROWS, K)
    dst2d = jnp.pad(dst, (0, pad)).reshape(NROWS, K)
    ew2d = jnp.pad(edge_weight, (0, pad)).reshape(NROWS, K)
    e3 = jnp.stack([src2d, dst2d], axis=1)
    b1r = b1.reshape(1, D_HID)
    b2r = b2.reshape(1, N_CLASSES)

    xp = jnp.pad(x, ((0, N_ACC - N_NODES), (0, 0)))

    degp = _sc_degree(dst2d, ew2d).reshape(NC, N_ACC)

    grid = N_ACC // _RB
    h1s = pl.pallas_call(
        _tc_mm1_body,
        grid=(grid,),
        in_specs=[
            pl.BlockSpec((_RB, D_IN), lambda i: (i, 0)),
            pl.BlockSpec((D_IN, D_HID), lambda i: (0, 0)),
            pl.BlockSpec((NC, _RB), lambda i: (0, i)),
        ],
        out_specs=pl.BlockSpec((_RB, D_HID), lambda i: (i, 0)),
        out_shape=jax.ShapeDtypeStruct((N_ACC, D_HID), jnp.float32),
    )(xp, W1, degp)

    acc1 = _sc_agg128(h1s, e3, ew2d)

    zs = pl.pallas_call(
        _tc_mid_body,
        grid=(grid,),
        in_specs=[
            pl.BlockSpec((1, _RB, D_HID), lambda i: (0, i, 0)),
            pl.BlockSpec((_RB, D_HID), lambda i: (i, 0)),
            pl.BlockSpec((NC, _RB), lambda i: (0, i)),
            pl.BlockSpec((1, D_HID), lambda i: (0, 0)),
        ],
        out_specs=pl.BlockSpec((_RB, D_HID), lambda i: (i, 0)),
        out_shape=jax.ShapeDtypeStruct((N_ACC, D_HID), jnp.float32),
    )(acc1, h1s, degp, b1r)

    acc2 = _sc_agg128(zs, e3, ew2d)

    out = pl.pallas_call(
        _tc_final_body,
        grid=(grid,),
        in_specs=[
            pl.BlockSpec((1, _RB, D_HID), lambda i: (0, i, 0)),
            pl.BlockSpec((_RB, D_HID), lambda i: (i, 0)),
            pl.BlockSpec((NC, _RB), lambda i: (0, i)),
            pl.BlockSpec((D_HID, N_CLASSES), lambda i: (0, 0)),
            pl.BlockSpec((1, N_CLASSES), lambda i: (0, 0)),
        ],
        out_specs=pl.BlockSpec((_RB, N_CLASSES), lambda i: (i, 0)),
        out_shape=jax.ShapeDtypeStruct((N_ACC, N_CLASSES), jnp.float32),
    )(acc2, zs, degp, W2, b2r)

    return out[:N_NODES]


# speedup vs baseline: 1.6643x; 1.6643x over previous
import functools

import jax
import jax.numpy as jnp
from jax import lax
from jax.experimental import pallas as pl
from jax.experimental.pallas import tpu as pltpu
from jax.experimental.pallas import tpu_sc as plsc

N_NODES = 10000
N_EDGES = 320000
D_IN = 128
D_HID = 128
N_CLASSES = 40
D2P = 48

NC = 2
NS = 16
NW = NC * NS
K = 128
RPW = 80
E_PAD = NW * RPW * K
NROWS = NW * RPW
N_ACC = 10240
CR0 = 144
CR1 = 160 - CR0

_MESH = plsc.VectorSubcoreMesh(core_axis_name="c", subcore_axis_name="s")


def _make_sc_degree():
    @functools.partial(
        pl.kernel,
        out_type=jax.ShapeDtypeStruct((NC * N_ACC,), jnp.float32),
        mesh=_MESH,
        scratch_types=[
            pltpu.VMEM((RPW, K), jnp.int32),
            pltpu.VMEM((RPW, K), jnp.float32),
            pltpu.VMEM((640,), jnp.float32),
            pltpu.VMEM_SHARED((N_ACC,), jnp.float32),
        ],
    )
    def deg_kernel(dst_hbm, ew_hbm, deg_hbm, dstv, ewv, zv, deg_sh):
        c = lax.axis_index("c")
        s = lax.axis_index("s")
        w = c * NS + s
        for i in range(640 // 16):
            zv[pl.ds(i * 16, 16)] = jnp.zeros((16,), jnp.float32)
        pltpu.sync_copy(zv, deg_sh.at[pl.ds(s * 640, 640)])
        plsc.subcore_barrier()
        pltpu.sync_copy(dst_hbm.at[pl.ds(w * RPW, RPW)], dstv)
        pltpu.sync_copy(ew_hbm.at[pl.ds(w * RPW, RPW)], ewv)

        def body(j, carry):
            pltpu.sync_copy(ewv.at[j], deg_sh.at[dstv.at[j]], add=True)
            return carry

        lax.fori_loop(0, RPW, body, 0)
        plsc.subcore_barrier()
        pltpu.sync_copy(deg_sh.at[pl.ds(s * 640, 640)],
                        deg_hbm.at[pl.ds(c * N_ACC + s * 640, 640)])

    return deg_kernel


def _make_sc_agg(d):
    @functools.partial(
        pl.kernel,
        out_type=jax.ShapeDtypeStruct((NC, N_ACC, d), jnp.float32),
        mesh=_MESH,
        scratch_types=[
            pltpu.VMEM((8, 2, K), jnp.int32),
            pltpu.VMEM((8, K), jnp.float32),
            pltpu.VMEM((8, 2, K), jnp.int32),
            pltpu.VMEM((8, K), jnp.float32),
            pltpu.VMEM((K, d), jnp.float32),
            pltpu.VMEM((K, d), jnp.float32),
            pltpu.VMEM_SHARED((N_ACC, d), jnp.float32),
            pltpu.SemaphoreType.DMA,
            pltpu.SemaphoreType.DMA,
            pltpu.SemaphoreType.DMA,
            pltpu.SemaphoreType.DMA,
            pltpu.SemaphoreType.DMA,
            pltpu.SemaphoreType.DMA,
        ],
    )
    def agg_kernel(h_hbm, e3_hbm, ew_hbm, acc_hbm,
                   e3A, ewA, e3B, ewB, rows0, rows1, acc_sh,
                   g0, g1, s0, s1, iA, iB):
        c = lax.axis_index("c")
        s = lax.axis_index("s")
        base_w = jnp.where(c == 0, s * CR0, NS * CR0 + s * CR1)
        nsup = jnp.where(c == 0, CR0 // 16, CR1 // 16)

        def zbody(i, carry):
            for t in range(d // 16):
                rows0[i, pl.ds(t * 16, 16)] = jnp.zeros((16,), jnp.float32)
            return carry

        lax.fori_loop(0, K, zbody, 0)
        for m in range(640 // K):
            pltpu.sync_copy(rows0, acc_sh.at[pl.ds(s * 640 + m * K, K)])
        plsc.subcore_barrier()

        def scale(buf, ewb, j):
            def group(g, carry2):
                w16 = ewb[j, pl.ds(g * 16, 16)]
                for l in range(16):
                    sc = w16[l]
                    k = g * 16 + l
                    for t in range(d // 16):
                        buf[k, pl.ds(t * 16, 16)] = (
                            buf[k, pl.ds(t * 16, 16)] * sc)
                return carry2

            lax.fori_loop(0, K // 16, group, 0)

        def wait_gather(buf, sem):
            pltpu.make_async_copy(h_hbm.at[e3A.at[0, 0]], buf, sem).wait()

        def issue_idx(base, e3b, ewb, sem):
            pltpu.async_copy(e3_hbm.at[pl.ds(base, 8)], e3b, sem)
            pltpu.async_copy(ew_hbm.at[pl.ds(base, 8)], ewb, sem)

        def wait_idx(e3b, ewb, sem):
            pltpu.make_async_copy(e3_hbm.at[pl.ds(0, 8)], e3b, sem).wait()
            pltpu.make_async_copy(ew_hbm.at[pl.ds(0, 8)], ewb, sem).wait()

        def run_block(e3b, ewb, e3n, ewn, sem_n, base_next2, load_next2,
                      has_next):
            def pair(p, carry1):
                j0 = 2 * p
                j1 = 2 * p + 1
                wait_gather(rows0, g0)
                scale(rows0, ewb, j0)
                d0 = pltpu.async_copy(rows0, acc_sh.at[e3b.at[j0, 1]], s0,
                                      add=True)
                wait_gather(rows1, g1)
                scale(rows1, ewb, j1)
                d1 = pltpu.async_copy(rows1, acc_sh.at[e3b.at[j1, 1]], s1,
                                      add=True)
                d0.wait()

                @pl.when(p < 3)
                def _():
                    pltpu.async_copy(h_hbm.at[e3b.at[j0 + 2, 0]], rows0, g0)

                d1.wait()

                @pl.when(p < 3)
                def _():
                    pltpu.async_copy(h_hbm.at[e3b.at[j1 + 2, 0]], rows1, g1)

                @pl.when((p == 3) & has_next)
                def _():
                    wait_idx(e3n, ewn, sem_n)
                    pltpu.async_copy(h_hbm.at[e3n.at[0, 0]], rows0, g0)
                    pltpu.async_copy(h_hbm.at[e3n.at[1, 0]], rows1, g1)

                @pl.when((p == 3) & load_next2)
                def _():
                    issue_idx(base_next2, e3b, ewb,
                              iA if e3b is e3A else iB)

                return carry1

            lax.fori_loop(0, 4, pair, 0)

        def sup(t, carry):
            base = base_w + t * 16
            not_last = t < nsup - 1
            run_block(e3A, ewA, e3B, ewB, iB, base + 16, not_last,
                      jnp.bool_(True))
            run_block(e3B, ewB, e3A, ewA, iA, base + 24, not_last, not_last)
            return carry

        issue_idx(base_w, e3A, ewA, iA)
        wait_idx(e3A, ewA, iA)
        issue_idx(base_w + 8, e3B, ewB, iB)
        pltpu.async_copy(h_hbm.at[e3A.at[0, 0]], rows0, g0)
        pltpu.async_copy(h_hbm.at[e3A.at[1, 0]], rows1, g1)
        lax.fori_loop(0, nsup, sup, 0)
        plsc.subcore_barrier()
        pltpu.sync_copy(acc_sh.at[pl.ds(s * 640, 640)],
                        acc_hbm.at[c, pl.ds(s * 640, 640)])

    return agg_kernel


_sc_degree = _make_sc_degree()
_sc_agg128 = _make_sc_agg(D_HID)


def _dis_of(degp_blk):
    deg = degp_blk[0, :] + degp_blk[1, :] + 1.0
    return lax.rsqrt(deg)


def _tc_mm1_body(x_ref, w_ref, degp_ref, out_ref):
    dis = _dis_of(degp_ref[...])
    h = jnp.dot(x_ref[...], w_ref[...], preferred_element_type=jnp.float32)
    out_ref[...] = h * dis[:, None]


def _tc_mid_body(acc_ref, h1s_ref, degp_ref, b1_ref, out_ref):
    dis = _dis_of(degp_ref[...])
    a = acc_ref[0] + acc_ref[1] + h1s_ref[...]
    z = jnp.maximum(a * dis[:, None] + b1_ref[...], 0.0)
    out_ref[...] = z * dis[:, None]


def _tc_final_body(acc_ref, zs_ref, degp_ref, w2_ref, b2_ref, out_ref):
    dis = _dis_of(degp_ref[...])
    a = (acc_ref[0] + acc_ref[1] + zs_ref[...]) * dis[:, None]
    out_ref[...] = jnp.dot(a, w2_ref[...],
                           preferred_element_type=jnp.float32) + b2_ref[...]


_RB = 1024


def kernel(x, edge_index, edge_weight, W1, b1, W2, b2):
    src = edge_index[0].astype(jnp.int32)
    dst = edge_index[1].astype(jnp.int32)
    pad = E_PAD - N_EDGES
    src2d = jnp.pad(src, (0, pad)).reshape(NROWS, K)
    dst2d = jnp.pad(dst, (0, pad)).reshape(NROWS, K)
    ew2d = jnp.pad(edge_weight, (0, pad)).reshape(NROWS, K)
    e3 = jnp.stack([src2d, dst2d], axis=1)
    b1r = b1.reshape(1, D_HID)
    b2r = b2.reshape(1, N_CLASSES)

    xp = jnp.pad(x, ((0, N_ACC - N_NODES), (0, 0)))

    degp = _sc_degree(dst2d, ew2d).reshape(NC, N_ACC)

    grid = N_ACC // _RB
    h1s = pl.pallas_call(
        _tc_mm1_body,
        grid=(grid,),
        in_specs=[
            pl.BlockSpec((_RB, D_IN), lambda i: (i, 0)),
            pl.BlockSpec((D_IN, D_HID), lambda i: (0, 0)),
            pl.BlockSpec((NC, _RB), lambda i: (0, i)),
        ],
        out_specs=pl.BlockSpec((_RB, D_HID), lambda i: (i, 0)),
        out_shape=jax.ShapeDtypeStruct((N_ACC, D_HID), jnp.float32),
    )(xp, W1, degp)

    acc1 = _sc_agg128(h1s, e3, ew2d)

    zs = pl.pallas_call(
        _tc_mid_body,
        grid=(grid,),
        in_specs=[
            pl.BlockSpec((NC, _RB, D_HID), lambda i: (0, i, 0)),
            pl.BlockSpec((_RB, D_HID), lambda i: (i, 0)),
            pl.BlockSpec((NC, _RB), lambda i: (0, i)),
            pl.BlockSpec((1, D_HID), lambda i: (0, 0)),
        ],
        out_specs=pl.BlockSpec((_RB, D_HID), lambda i: (i, 0)),
        out_shape=jax.ShapeDtypeStruct((N_ACC, D_HID), jnp.float32),
    )(acc1, h1s, degp, b1r)

    acc2 = _sc_agg128(zs, e3, ew2d)

    out = pl.pallas_call(
        _tc_final_body,
        grid=(grid,),
        in_specs=[
            pl.BlockSpec((NC, _RB, D_HID), lambda i: (0, i, 0)),
            pl.BlockSpec((_RB, D_HID), lambda i: (i, 0)),
            pl.BlockSpec((NC, _RB), lambda i: (0, i)),
            pl.BlockSpec((D_HID, N_CLASSES), lambda i: (0, 0)),
            pl.BlockSpec((1, N_CLASSES), lambda i: (0, 0)),
        ],
        out_specs=pl.BlockSpec((_RB, N_CLASSES), lambda i: (i, 0)),
        out_shape=jax.ShapeDtypeStruct((N_ACC, N_CLASSES), jnp.float32),
    )(acc2, zs, degp, W2, b2r)

    return out[:N_NODES]
